# baseline (speedup 1.0000x reference)
import functools

import jax
import jax.numpy as jnp
from jax import lax
from jax.experimental import pallas as pl
from jax.experimental.pallas import tpu as pltpu
from jax.experimental.pallas import tpu_sc as plsc

N_NODES = 10000
D = 128

NC = 2
NS = 16
NW = NC * NS

C = 64
NR = 10112
ROWS_PER_TEC = NR // NS

E_TOTAL = 320000 + N_NODES
EPW = ((E_TOTAL + NW * C - 1) // (NW * C)) * C
NCHUNK = EPW // C
E_PAD = EPW * NW
E_ALLOC = E_PAD + 2 * C


_RB = 1000


def _transform_body(x_ref, wl_ref, wr_ref, xl_ref, xr_ref):
    x = x_ref[...]
    xl_ref[...] = jnp.dot(x, wl_ref[...], preferred_element_type=jnp.float32)
    xr_ref[...] = jnp.dot(x, wr_ref[...], preferred_element_type=jnp.float32)


def _tc_transform(x, Wl, Wr):
    n = x.shape[0]
    grid = n // _RB
    return pl.pallas_call(
        _transform_body,
        grid=(grid,),
        in_specs=[
            pl.BlockSpec((_RB, D), lambda i: (i, 0)),
            pl.BlockSpec((D, D), lambda i: (0, 0)),
            pl.BlockSpec((D, D), lambda i: (0, 0)),
        ],
        out_specs=[
            pl.BlockSpec((_RB, D), lambda i: (i, 0)),
            pl.BlockSpec((_RB, D), lambda i: (i, 0)),
        ],
        out_shape=[
            jax.ShapeDtypeStruct((n, D), jnp.float32),
            jax.ShapeDtypeStruct((n, D), jnp.float32),
        ],
    )(x, Wl, Wr)


def _combine_transform_body(num_ref, den_ref, b_ref, wl_ref, wr_ref,
                            xl_ref, xr_ref):
    num = num_ref[0] + num_ref[1]
    den = den_ref[0] + den_ref[1]
    h = jnp.maximum(num / (den + 1e-16) + b_ref[...][None, :], 0.0)
    xl_ref[...] = jnp.dot(h, wl_ref[...], preferred_element_type=jnp.float32)
    xr_ref[...] = jnp.dot(h, wr_ref[...], preferred_element_type=jnp.float32)


def _tc_combine_transform(num, den, b, Wl, Wr):
    grid = N_NODES // _RB
    return pl.pallas_call(
        _combine_transform_body,
        grid=(grid,),
        in_specs=[
            pl.BlockSpec((NC, _RB, D), lambda i: (0, i, 0)),
            pl.BlockSpec((NC, _RB, 1), lambda i: (0, i, 0)),
            pl.BlockSpec((D,), lambda i: (0,)),
            pl.BlockSpec((D, D), lambda i: (0, 0)),
            pl.BlockSpec((D, D), lambda i: (0, 0)),
        ],
        out_specs=[
            pl.BlockSpec((_RB, D), lambda i: (i, 0)),
            pl.BlockSpec((_RB, D), lambda i: (i, 0)),
        ],
        out_shape=[
            jax.ShapeDtypeStruct((N_NODES, D), jnp.float32),
            jax.ShapeDtypeStruct((N_NODES, D), jnp.float32),
        ],
    )(num, den, b, Wl, Wr)


def _final_body(num_ref, den_ref, b_ref, y_ref):
    num = num_ref[0] + num_ref[1]
    den = den_ref[0] + den_ref[1]
    y_ref[...] = jnp.maximum(num / (den + 1e-16) + b_ref[...][None, :], 0.0)


def _tc_final(num, den, b):
    grid = N_NODES // _RB
    return pl.pallas_call(
        _final_body,
        grid=(grid,),
        in_specs=[
            pl.BlockSpec((NC, _RB, D), lambda i: (0, i, 0)),
            pl.BlockSpec((NC, _RB, 1), lambda i: (0, i, 0)),
            pl.BlockSpec((D,), lambda i: (0,)),
        ],
        out_specs=pl.BlockSpec((_RB, D), lambda i: (i, 0)),
        out_shape=jax.ShapeDtypeStruct((N_NODES, D), jnp.float32),
    )(num, den, b)


DR = 80


@functools.cache
def _make_sc_edge_pass():
  @functools.partial(
    pl.kernel,
    out_type=(
        jax.ShapeDtypeStruct((NC, NR, D), jnp.float32),
        jax.ShapeDtypeStruct((NC, DR, D), jnp.float32),
    ),
    mesh=plsc.VectorSubcoreMesh(core_axis_name="c", subcore_axis_name="s",
                                num_cores=NC, num_subcores=NS),
    compiler_params=pltpu.CompilerParams(needs_layout_passes=False),
    scratch_types=[
        pltpu.VMEM_SHARED((NR, D), jnp.float32),
        pltpu.VMEM_SHARED((DR, D), jnp.float32),
        pltpu.VMEM((2, C), jnp.int32),
        pltpu.VMEM((2, C), jnp.int32),
        pltpu.VMEM((2, C), jnp.int32),
        pltpu.VMEM((2, C, D), jnp.float32),
        pltpu.VMEM((2, C, D), jnp.float32),
        pltpu.VMEM((DR, D), jnp.float32),
        pltpu.VMEM((DR,), jnp.int32),
        pltpu.VMEM((D,), jnp.float32),
        pltpu.VMEM((272,), jnp.float32),
        pltpu.SemaphoreType.DMA,
        pltpu.SemaphoreType.DMA,
        pltpu.SemaphoreType.DMA,
        pltpu.SemaphoreType.DMA,
        pltpu.SemaphoreType.DMA,
        pltpu.SemaphoreType.DMA,
        pltpu.SemaphoreType.DMA,
        pltpu.SemaphoreType.DMA,
    ],
  )
  def _sc_edge_pass(xl_hbm, xr_hbm, att_hbm, src_hbm, dst_hbm,
                    num_hbm, den_hbm,
                    table, dent, srcv, dstv, dsts, xlv, xrv, denv, rowids,
                    attv, rbuf,
                    si0, si1, sgl0, sgl1, sgr0, sgr1, ssc0, ssc1):
    cid = lax.axis_index("c")
    sid = lax.axis_index("s")
    wid = sid * NC + cid

    si = (si0, si1)
    sgl = (sgl0, sgl1)
    sgr = (sgr0, sgr1)
    ssc = (ssc0, ssc1)

    zero16 = jnp.zeros((16,), jnp.float32)
    iota16 = jnp.arange(16, dtype=jnp.int32)
    base0 = wid * EPW

    def zrow(r, carry):
        for kk in range(D // 16):
            xlv[0, r, pl.ds(kk * 16, 16)] = zero16
        return carry

    lax.fori_loop(0, C, zrow, 0)
    for j in range(ROWS_PER_TEC // C):
        pltpu.sync_copy(xlv.at[0],
                        table.at[pl.ds(sid * ROWS_PER_TEC + j * C, C)])
    _rem = ROWS_PER_TEC % C
    if _rem:
        pltpu.sync_copy(
            xlv.at[0, pl.ds(0, _rem)],
            table.at[pl.ds(sid * ROWS_PER_TEC + (ROWS_PER_TEC // C) * C,
                           _rem)])

    def zden(r, carry):
        for kk in range(D // 16):
            denv[r, pl.ds(kk * 16, 16)] = zero16
        return carry

    lax.fori_loop(0, DR, zden, 0)

    @pl.when(sid == 0)
    def _():
        pltpu.sync_copy(xlv.at[0], dent.at[pl.ds(0, C)])
        pltpu.sync_copy(xlv.at[0, pl.ds(0, DR - C)], dent.at[pl.ds(C, DR - C)])

    def rowid_init(t, carry):
        rowids[pl.ds(t * 16, 16)] = iota16 + t * 16
        return carry

    lax.fori_loop(0, DR // 16, rowid_init, 0)
    pltpu.sync_copy(att_hbm, attv)
    plsc.subcore_barrier()

    attks = [attv[pl.ds(kk * 16, 16)] for kk in range(8)]
    iota17 = iota16 * 17

    def idx_start(c, b):
        pltpu.make_async_copy(
            src_hbm.at[pl.ds(base0 + c * C, C)], srcv.at[b], si[b]).start()
        pltpu.make_async_copy(
            dst_hbm.at[pl.ds(base0 + c * C, C)], dstv.at[b], si[b]).start()

    def idx_wait(b):
        pltpu.make_async_copy(
            src_hbm.at[pl.ds(0, C)], srcv.at[b], si[b]).wait()
        pltpu.make_async_copy(
            dst_hbm.at[pl.ds(0, C)], dstv.at[b], si[b]).wait()

    def gather_start(b):
        pltpu.make_async_copy(xl_hbm.at[srcv.at[b]], xlv.at[b],
                              sgl[b]).start()
        pltpu.make_async_copy(xr_hbm.at[dstv.at[b]], xrv.at[b],
                              sgr[b]).start()

    def gather_wait(b):
        pltpu.make_async_copy(xl_hbm.at[srcv.at[b]], xlv.at[b],
                              sgl[b]).wait()
        pltpu.make_async_copy(xr_hbm.at[dstv.at[b]], xrv.at[b],
                              sgr[b]).wait()

    def scatter_start(b):
        pltpu.make_async_copy(xlv.at[b], table.at[dsts.at[b]],
                              ssc[b]).start(add=True)

    def scatter_wait(b):
        pltpu.make_async_copy(xlv.at[b], table.at[dsts.at[b]],
                              ssc[b]).wait()

    def save_dst(b):
        for t in range(C // 16):
            dsts[b, pl.ds(t * 16, 16)] = dstv[b, pl.ds(t * 16, 16)]

    def compute(b):
        def grp_body(g, gcarry):
            base_e = g * 16

            def blk(blki, evg):
                for jj in range(4):
                    j = blki * 4 + jj
                    e = base_e + j
                    avs = []
                    ms = []
                    for kk in range(8):
                        a = xlv[b, e, pl.ds(kk * 16, 16)]
                        bb = xrv[b, e, pl.ds(kk * 16, 16)]
                        avs.append(a)
                        t = a + bb
                        t = jnp.maximum(t, 0.2 * t)
                        ms.append(t * attks[kk])
                    while len(ms) > 1:
                        ms = [ms[i] + ms[i + 1]
                              for i in range(0, len(ms), 2)]
                    s = plsc.cumsum(ms[0])[15]
                    ev = jnp.exp(jnp.full((16,), s, jnp.float32))
                    evg = jnp.where(iota16 == j, ev, evg)
                    for kk in range(8):
                        xlv[b, e, pl.ds(kk * 16, 16)] = avs[kk] * ev
                return evg

            evg = lax.fori_loop(0, 4, blk, zero16)
            dstg = dsts[b, pl.ds(base_e, 16)]
            rhi = lax.shift_right_logical(dstg, 7)
            rlo = jnp.bitwise_and(dstg, 127)
            for j in range(16):
                plsc.addupdate_scatter(denv, [rhi, rlo], evg,
                                       mask=iota16 == j)
            return gcarry

        lax.fori_loop(0, C // 16, grp_body, 0)

    pltpu.sync_copy(src_hbm.at[pl.ds(base0, C)], srcv.at[0])
    pltpu.sync_copy(dst_hbm.at[pl.ds(base0, C)], dstv.at[0])
    gather_start(0)
    idx_start(1, 1)

    gather_wait(0)
    save_dst(0)
    idx_start(2, 0)
    idx_wait(1)
    gather_start(1)
    compute(0)
    scatter_start(0)

    gather_wait(1)
    save_dst(1)
    idx_start(3, 1)
    scatter_wait(0)
    idx_wait(0)
    gather_start(0)
    compute(1)
    scatter_start(1)

    def pair_body(i2, carry):
        gather_wait(0)
        save_dst(0)
        idx_start(2 * i2 + 2, 0)
        scatter_wait(1)
        idx_wait(1)
        gather_start(1)
        compute(0)
        scatter_start(0)
        gather_wait(1)
        save_dst(1)
        idx_start(2 * i2 + 3, 1)
        scatter_wait(0)
        idx_wait(0)
        gather_start(0)
        compute(1)
        scatter_start(1)
        return carry

    lax.fori_loop(1, NCHUNK // 2, pair_body, 0)

    gather_wait(0)
    idx_wait(1)
    scatter_wait(1)

    pltpu.sync_copy(denv, dent.at[rowids], add=True)
    plsc.subcore_barrier()
    pltpu.sync_copy(
        table.at[pl.ds(sid * ROWS_PER_TEC, ROWS_PER_TEC)],
        num_hbm.at[cid, pl.ds(sid * ROWS_PER_TEC, ROWS_PER_TEC)],
    )

    @pl.when(sid == 0)
    def _():
        pltpu.sync_copy(dent, den_hbm.at[cid])

  return _sc_edge_pass


def kernel(node_features, Wl1, Wr1, att1, b1, Wl2, Wr2, att2, b2, edge_index):
    n = node_features.shape[0]
    loop = jnp.arange(n, dtype=jnp.int32)
    src = jnp.concatenate([
        edge_index[0].astype(jnp.int32), loop,
        jnp.zeros((E_ALLOC - E_TOTAL,), jnp.int32),
    ])
    dst = jnp.concatenate([
        edge_index[1].astype(jnp.int32), loop,
        jnp.full((E_PAD - E_TOTAL,), N_NODES, jnp.int32),
        jnp.zeros((E_ALLOC - E_PAD,), jnp.int32),
    ])

    sc_edge_pass = _make_sc_edge_pass()
    xl1, xr1 = _tc_transform(node_features, Wl1, Wr1)
    num1, den1 = sc_edge_pass(xl1, xr1, att1, src, dst)
    den1 = den1.reshape(NC, DR * D)[:, :N_NODES, None]
    xl2, xr2 = _tc_combine_transform(num1[:, :N_NODES, :], den1, b1, Wl2, Wr2)
    num2, den2 = sc_edge_pass(xl2, xr2, att2, src, dst)
    den2 = den2.reshape(NC, DR * D)[:, :N_NODES, None]
    return _tc_final(num2[:, :N_NODES, :], den2, b2)


# speedup vs baseline: 1.1932x; 1.1932x over previous
import functools

import jax
import jax.numpy as jnp
from jax import lax
from jax.experimental import pallas as pl
from jax.experimental.pallas import tpu as pltpu
from jax.experimental.pallas import tpu_sc as plsc

N_NODES = 10000
D = 128

NC = 2
NS = 16
NW = NC * NS

C = 64
NR = 10112
ROWS_PER_TEC = NR // NS

E_TOTAL = 320000 + N_NODES
EPW = ((E_TOTAL + NW * C - 1) // (NW * C)) * C
NCHUNK = EPW // C
E_PAD = EPW * NW
E_ALLOC = E_PAD + 2 * C


_RB = 1000


def _transform_body(x_ref, wl_ref, wr_ref, xl_ref, xr_ref):
    x = x_ref[...]
    xl_ref[...] = jnp.dot(x, wl_ref[...], preferred_element_type=jnp.float32)
    xr_ref[...] = jnp.dot(x, wr_ref[...], preferred_element_type=jnp.float32)


def _tc_transform(x, Wl, Wr):
    n = x.shape[0]
    grid = n // _RB
    return pl.pallas_call(
        _transform_body,
        grid=(grid,),
        in_specs=[
            pl.BlockSpec((_RB, D), lambda i: (i, 0)),
            pl.BlockSpec((D, D), lambda i: (0, 0)),
            pl.BlockSpec((D, D), lambda i: (0, 0)),
        ],
        out_specs=[
            pl.BlockSpec((_RB, D), lambda i: (i, 0)),
            pl.BlockSpec((_RB, D), lambda i: (i, 0)),
        ],
        out_shape=[
            jax.ShapeDtypeStruct((n, D), jnp.float32),
            jax.ShapeDtypeStruct((n, D), jnp.float32),
        ],
    )(x, Wl, Wr)


def _combine_transform_body(num_ref, den_ref, b_ref, wl_ref, wr_ref,
                            xl_ref, xr_ref):
    num = num_ref[0] + num_ref[1]
    den = den_ref[0] + den_ref[1]
    h = jnp.maximum(num / (den + 1e-16) + b_ref[...][None, :], 0.0)
    xl_ref[...] = jnp.dot(h, wl_ref[...], preferred_element_type=jnp.float32)
    xr_ref[...] = jnp.dot(h, wr_ref[...], preferred_element_type=jnp.float32)


def _tc_combine_transform(num, den, b, Wl, Wr):
    grid = N_NODES // _RB
    return pl.pallas_call(
        _combine_transform_body,
        grid=(grid,),
        in_specs=[
            pl.BlockSpec((NC, _RB, D), lambda i: (0, i, 0)),
            pl.BlockSpec((NC, _RB, 1), lambda i: (0, i, 0)),
            pl.BlockSpec((D,), lambda i: (0,)),
            pl.BlockSpec((D, D), lambda i: (0, 0)),
            pl.BlockSpec((D, D), lambda i: (0, 0)),
        ],
        out_specs=[
            pl.BlockSpec((_RB, D), lambda i: (i, 0)),
            pl.BlockSpec((_RB, D), lambda i: (i, 0)),
        ],
        out_shape=[
            jax.ShapeDtypeStruct((N_NODES, D), jnp.float32),
            jax.ShapeDtypeStruct((N_NODES, D), jnp.float32),
        ],
    )(num, den, b, Wl, Wr)


def _final_body(num_ref, den_ref, b_ref, y_ref):
    num = num_ref[0] + num_ref[1]
    den = den_ref[0] + den_ref[1]
    y_ref[...] = jnp.maximum(num / (den + 1e-16) + b_ref[...][None, :], 0.0)


def _tc_final(num, den, b):
    grid = N_NODES // _RB
    return pl.pallas_call(
        _final_body,
        grid=(grid,),
        in_specs=[
            pl.BlockSpec((NC, _RB, D), lambda i: (0, i, 0)),
            pl.BlockSpec((NC, _RB, 1), lambda i: (0, i, 0)),
            pl.BlockSpec((D,), lambda i: (0,)),
        ],
        out_specs=pl.BlockSpec((_RB, D), lambda i: (i, 0)),
        out_shape=jax.ShapeDtypeStruct((N_NODES, D), jnp.float32),
    )(num, den, b)


DR = 80


@functools.cache
def _make_sc_edge_pass():
  @functools.partial(
    pl.kernel,
    out_type=(
        jax.ShapeDtypeStruct((NC, NR, D), jnp.float32),
        jax.ShapeDtypeStruct((NC, DR, D), jnp.float32),
    ),
    mesh=plsc.VectorSubcoreMesh(core_axis_name="c", subcore_axis_name="s",
                                num_cores=NC, num_subcores=NS),
    compiler_params=pltpu.CompilerParams(needs_layout_passes=False),
    scratch_types=[
        pltpu.VMEM_SHARED((NR, D), jnp.float32),
        pltpu.VMEM_SHARED((DR, D), jnp.float32),
        pltpu.VMEM((2, C), jnp.int32),
        pltpu.VMEM((2, C), jnp.int32),
        pltpu.VMEM((2, C), jnp.int32),
        pltpu.VMEM((2, C, D), jnp.float32),
        pltpu.VMEM((2, C, D), jnp.float32),
        pltpu.VMEM((DR, D), jnp.float32),
        pltpu.VMEM((DR,), jnp.int32),
        pltpu.VMEM((D,), jnp.float32),
        pltpu.VMEM((272,), jnp.float32),
        pltpu.SemaphoreType.DMA,
        pltpu.SemaphoreType.DMA,
        pltpu.SemaphoreType.DMA,
        pltpu.SemaphoreType.DMA,
        pltpu.SemaphoreType.DMA,
        pltpu.SemaphoreType.DMA,
        pltpu.SemaphoreType.DMA,
        pltpu.SemaphoreType.DMA,
    ],
  )
  def _sc_edge_pass(xl_hbm, xr_hbm, att_hbm, src_hbm, dst_hbm,
                    num_hbm, den_hbm,
                    table, dent, srcv, dstv, dsts, xlv, xrv, denv, rowids,
                    attv, rbuf,
                    si0, si1, sgl0, sgl1, sgr0, sgr1, ssc0, ssc1):
    cid = lax.axis_index("c")
    sid = lax.axis_index("s")
    wid = sid * NC + cid

    si = (si0, si1)
    sgl = (sgl0, sgl1)
    sgr = (sgr0, sgr1)
    ssc = (ssc0, ssc1)

    zero16 = jnp.zeros((16,), jnp.float32)
    iota16 = jnp.arange(16, dtype=jnp.int32)
    base0 = wid * EPW

    def zrow(r, carry):
        for kk in range(D // 16):
            xlv[0, r, pl.ds(kk * 16, 16)] = zero16
        return carry

    lax.fori_loop(0, C, zrow, 0)
    for j in range(ROWS_PER_TEC // C):
        pltpu.sync_copy(xlv.at[0],
                        table.at[pl.ds(sid * ROWS_PER_TEC + j * C, C)])
    _rem = ROWS_PER_TEC % C
    if _rem:
        pltpu.sync_copy(
            xlv.at[0, pl.ds(0, _rem)],
            table.at[pl.ds(sid * ROWS_PER_TEC + (ROWS_PER_TEC // C) * C,
                           _rem)])

    def zden(r, carry):
        for kk in range(D // 16):
            denv[r, pl.ds(kk * 16, 16)] = zero16
        return carry

    lax.fori_loop(0, DR, zden, 0)

    @pl.when(sid == 0)
    def _():
        pltpu.sync_copy(xlv.at[0], dent.at[pl.ds(0, C)])
        pltpu.sync_copy(xlv.at[0, pl.ds(0, DR - C)], dent.at[pl.ds(C, DR - C)])

    def rowid_init(t, carry):
        rowids[pl.ds(t * 16, 16)] = iota16 + t * 16
        return carry

    lax.fori_loop(0, DR // 16, rowid_init, 0)
    pltpu.sync_copy(att_hbm, attv)
    plsc.subcore_barrier()

    attks = [attv[pl.ds(kk * 16, 16)] for kk in range(8)]
    iota17 = iota16 * 17

    def idx_start(c, b):
        pltpu.make_async_copy(
            src_hbm.at[pl.ds(base0 + c * C, C)], srcv.at[b], si[b]).start()
        pltpu.make_async_copy(
            dst_hbm.at[pl.ds(base0 + c * C, C)], dstv.at[b], si[b]).start()

    def idx_wait(b):
        pltpu.make_async_copy(
            src_hbm.at[pl.ds(0, C)], srcv.at[b], si[b]).wait()
        pltpu.make_async_copy(
            dst_hbm.at[pl.ds(0, C)], dstv.at[b], si[b]).wait()

    def gather_start(b):
        pltpu.make_async_copy(xl_hbm.at[srcv.at[b]], xlv.at[b],
                              sgl[b]).start()
        pltpu.make_async_copy(xr_hbm.at[dstv.at[b]], xrv.at[b],
                              sgr[b]).start()

    def gather_wait(b):
        pltpu.make_async_copy(xl_hbm.at[srcv.at[b]], xlv.at[b],
                              sgl[b]).wait()
        pltpu.make_async_copy(xr_hbm.at[dstv.at[b]], xrv.at[b],
                              sgr[b]).wait()

    def scatter_start(b):
        pltpu.make_async_copy(xlv.at[b], table.at[dsts.at[b]],
                              ssc[b]).start(add=True)

    def scatter_wait(b):
        pltpu.make_async_copy(xlv.at[b], table.at[dsts.at[b]],
                              ssc[b]).wait()

    def save_dst(b):
        for t in range(C // 16):
            dsts[b, pl.ds(t * 16, 16)] = dstv[b, pl.ds(t * 16, 16)]

    def compute(b):
        def grp_body(g, gcarry):
            base_e = g * 16
            for j in range(16):
                e = base_e + j
                ms = []
                for kk in range(8):
                    a = xlv[b, e, pl.ds(kk * 16, 16)]
                    bb = xrv[b, e, pl.ds(kk * 16, 16)]
                    t = a + bb
                    t = jnp.maximum(t, 0.2 * t)
                    ms.append(t * attks[kk])
                while len(ms) > 1:
                    ms = [ms[i] + ms[i + 1] for i in range(0, len(ms), 2)]
                rbuf[pl.ds(j * 17, 16)] = ms[0]
            acc = zero16
            for k in range(16):
                col = plsc.load_gather(rbuf, [iota17 + k])
                acc = acc + col
            evg = jnp.exp(acc)
            dstg = dsts[b, pl.ds(base_e, 16)]
            rhi = lax.shift_right_logical(dstg, 7)
            rlo = jnp.bitwise_and(dstg, 127)
            for j in range(16):
                plsc.addupdate_scatter(denv, [rhi, rlo], evg,
                                       mask=iota16 == j)
            for j in range(16):
                e = base_e + j
                s = evg[j]
                for kk in range(8):
                    xlv[b, e, pl.ds(kk * 16, 16)] = (
                        xlv[b, e, pl.ds(kk * 16, 16)] * s)
            return gcarry

        lax.fori_loop(0, C // 16, grp_body, 0)

    pltpu.sync_copy(src_hbm.at[pl.ds(base0, C)], srcv.at[0])
    pltpu.sync_copy(dst_hbm.at[pl.ds(base0, C)], dstv.at[0])
    gather_start(0)
    idx_start(1, 1)

    gather_wait(0)
    save_dst(0)
    idx_start(2, 0)
    idx_wait(1)
    gather_start(1)
    compute(0)
    scatter_start(0)

    gather_wait(1)
    save_dst(1)
    idx_start(3, 1)
    scatter_wait(0)
    idx_wait(0)
    gather_start(0)
    compute(1)
    scatter_start(1)

    def pair_body(i2, carry):
        gather_wait(0)
        save_dst(0)
        idx_start(2 * i2 + 2, 0)
        scatter_wait(1)
        idx_wait(1)
        gather_start(1)
        compute(0)
        scatter_start(0)
        gather_wait(1)
        save_dst(1)
        idx_start(2 * i2 + 3, 1)
        scatter_wait(0)
        idx_wait(0)
        gather_start(0)
        compute(1)
        scatter_start(1)
        return carry

    lax.fori_loop(1, NCHUNK // 2, pair_body, 0)

    gather_wait(0)
    idx_wait(1)
    scatter_wait(1)

    pltpu.sync_copy(denv, dent.at[rowids], add=True)
    plsc.subcore_barrier()
    pltpu.sync_copy(
        table.at[pl.ds(sid * ROWS_PER_TEC, ROWS_PER_TEC)],
        num_hbm.at[cid, pl.ds(sid * ROWS_PER_TEC, ROWS_PER_TEC)],
    )

    @pl.when(sid == 0)
    def _():
        pltpu.sync_copy(dent, den_hbm.at[cid])

  return _sc_edge_pass


def kernel(node_features, Wl1, Wr1, att1, b1, Wl2, Wr2, att2, b2, edge_index):
    n = node_features.shape[0]
    loop = jnp.arange(n, dtype=jnp.int32)
    src = jnp.concatenate([
        edge_index[0].astype(jnp.int32), loop,
        jnp.zeros((E_ALLOC - E_TOTAL,), jnp.int32),
    ])
    dst = jnp.concatenate([
        edge_index[1].astype(jnp.int32), loop,
        jnp.full((E_PAD - E_TOTAL,), N_NODES, jnp.int32),
        jnp.zeros((E_ALLOC - E_PAD,), jnp.int32),
    ])

    sc_edge_pass = _make_sc_edge_pass()
    xl1, xr1 = _tc_transform(node_features, Wl1, Wr1)
    num1, den1 = sc_edge_pass(xl1, xr1, att1, src, dst)
    den1 = den1.reshape(NC, DR * D)[:, :N_NODES, None]
    xl2, xr2 = _tc_combine_transform(num1[:, :N_NODES, :], den1, b1, Wl2, Wr2)
    num2, den2 = sc_edge_pass(xl2, xr2, att2, src, dst)
    den2 = den2.reshape(NC, DR * D)[:, :N_NODES, None]
    return _tc_final(num2[:, :N_NODES, :], den2, b2)
